# baseline (speedup 1.0000x reference)
import functools

import jax
import jax.numpy as jnp
from jax import lax
from jax.experimental import pallas as pl
from jax.experimental.pallas import tpu as pltpu
from jax.experimental.pallas import tpu_sc as plsc

BATCH = 16384
WIDTH = 64
ROWS = 1000000

_info = plsc.get_sparse_core_info()
_NC, _NS, _L = _info.num_cores, _info.num_subcores, _info.num_lanes
_NW = _NC * _NS
_WROWS = 512
_NWIN_FULL = 1953
_CAP = 3072
_YCHUNK = 2048


def _make_gather():
  mesh = plsc.VectorSubcoreMesh(core_axis_name="c", subcore_axis_name="s")

  @functools.partial(
      pl.kernel,
      mesh=mesh,
      compiler_params=pltpu.CompilerParams(needs_layout_passes=False),
      out_type=jax.ShapeDtypeStruct((BATCH * WIDTH,), jnp.float32),
      scratch_types=[
          pltpu.VMEM((_WROWS, WIDTH), jnp.float32),
          pltpu.VMEM((_YCHUNK,), jnp.int32),
          pltpu.VMEM((_CAP,), jnp.int32),
          pltpu.VMEM((_CAP,), jnp.int32),
          pltpu.VMEM((64,), jnp.int32),
          pltpu.VMEM((64,), jnp.int32),
          pltpu.VMEM((1024,), jnp.float32),
          pltpu.VMEM((1024,), jnp.float32),
          pltpu.VMEM((8, 128), jnp.int32),
          pltpu.VMEM((8, 128), jnp.int32),
          pltpu.VMEM((WIDTH,), jnp.int32),
          pltpu.SMEM((72,), jnp.int32),
          pltpu.SemaphoreType.DMA,
          pltpu.SemaphoreType.DMA,
          pltpu.SemaphoreType.DMA,
      ],
  )
  def gather_kernel(y_hbm, table_hbm, out_hbm, wbuf, ybuf, ylist, plist,
                    hist, starts, sb0, sb1, ib0, ib1, ibs, sm, semw, ss0, ss1):
    w = lax.axis_index("s") * _NC + lax.axis_index("c")
    lanes = lax.iota(jnp.int32, _L)
    ones = jnp.ones((_L,), jnp.int32)

    for v in range(4):
      hist[pl.ds(v * _L, _L)] = jnp.zeros((_L,), jnp.int32)

    def hist_chunk(c, _):
      pltpu.sync_copy(y_hbm.at[pl.ds(c * _YCHUNK, _YCHUNK)], ybuf)
      for v in range(_YCHUNK // _L):
        yv = ybuf[pl.ds(v * _L, _L)]
        m = (lax.shift_right_logical(yv, 9) & 31) == w
        lw = lax.shift_right_logical(yv, 14)
        plsc.addupdate_scatter(hist, [lw], ones, mask=m)
      return _

    lax.fori_loop(0, BATCH // _YCHUNK, hist_chunk, 0)

    carry = jnp.int32(0)
    for v in range(4):
      h = hist[pl.ds(v * _L, _L)]
      incl = plsc.cumsum(h)
      excl = incl - h + carry
      starts[pl.ds(v * _L, _L)] = excl
      for u in range(_L):
        sm[v * _L + u] = excl[u]
      carry = carry + incl[_L - 1]
    total = carry
    sm[64] = total
    sm[65] = 0
    sm[66] = 0

    @pl.when(total <= _CAP)
    def _fast():
      def compact_chunk(c, _):
        pltpu.sync_copy(y_hbm.at[pl.ds(c * _YCHUNK, _YCHUNK)], ybuf)
        for v in range(_YCHUNK // _L):
          yv = ybuf[pl.ds(v * _L, _L)]
          m = (lax.shift_right_logical(yv, 9) & 31) == w
          lw = lax.shift_right_logical(yv, 14)
          posv = lanes + (c * _YCHUNK + v * _L)
          rc, lastm = plsc.scan_count(lw, m)
          offs = plsc.load_gather(starts, [lw])
          dest = offs + rc - 1
          plsc.store_scatter(ylist, [dest], yv, mask=m)
          plsc.store_scatter(plist, [dest], posv, mask=m)
          plsc.addupdate_scatter(starts, [lw], rc, mask=m & lastm)
        return _

      lax.fori_loop(0, BATCH // _YCHUNK, compact_chunk, 0)

      def drain(sem, n):
        def body(i, _):
          pltpu.make_async_copy(out_hbm.at[pl.ds(0, 1024)], sb0, sem).wait()
          return _
        lax.fori_loop(0, n, body, 0)

      def do_batches(lo, hi):
        nb = (hi - lo + (_L - 1)) >> 4

        def build_issue(j, sb, ib, ss):
          ll = lo + j * _L + lanes
          li = jnp.minimum(ll, hi - 1)
          yv = plsc.load_gather(ylist, [li])
          pv = plsc.load_gather(plist, [li])
          localv = yv & (_WROWS - 1)
          pbase = lax.shift_left(pv, 6)
          for col in range(WIDTH):
            fullc = jnp.full((_L,), col, jnp.int32)
            sb[pl.ds(col * _L, _L)] = plsc.load_gather(wbuf, [localv, fullc])
            ib[col >> 3, pl.ds((col & 7) * _L, _L)] = pbase + col
          for q in range(8):
            pltpu.async_copy(sb.at[pl.ds(q * 128, 128)],
                             out_hbm.at[ib.at[q]], ss)

        def pair(i, _):
          @pl.when(2 * i < nb)
          def _a():
            @pl.when(i >= 1)
            def _wa():
              pltpu.make_async_copy(
                  out_hbm.at[pl.ds(0, 1024)], sb0, ss0).wait()
            build_issue(2 * i, sb0, ib0, ss0)
            sm[65] = 1

          @pl.when(2 * i + 1 < nb)
          def _b():
            @pl.when(i >= 1)
            def _wb():
              pltpu.make_async_copy(
                  out_hbm.at[pl.ds(0, 1024)], sb1, ss1).wait()
            build_issue(2 * i + 1, sb1, ib1, ss1)
            sm[66] = 1

          return _

        lax.fori_loop(0, (nb + 1) >> 1, pair, 0)

      def window_body(k, nrows):
        lo = sm[k]
        hi = sm[k + 1]

        @pl.when(hi > lo)
        def _():
          drain(ss0, sm[65])
          drain(ss1, sm[66])
          sm[65] = 0
          sm[66] = 0
          g = w + 32 * k
          pltpu.sync_copy(table_hbm.at[pl.ds(g * _WROWS, nrows)],
                          wbuf.at[pl.ds(0, nrows)])
          do_batches(lo, hi)

      def win_loop(k, _):
        window_body(k, _WROWS)
        return _

      nwin = 61 + jnp.where(w == 0, 1, 0)
      lax.fori_loop(0, nwin, win_loop, 0)

      @pl.when(w == 1)
      def _tail():
        window_body(jnp.int32(61), ROWS - _NWIN_FULL * _WROWS)

      drain(ss0, sm[65])
      drain(ss1, sm[66])

    @pl.when(total > _CAP)
    def _slow():
      def slow_chunk(c, _):
        pltpu.sync_copy(y_hbm.at[pl.ds(c * _YCHUNK, _YCHUNK)], ybuf)

        def vec_body(v, _):
          yv = ybuf[pl.ds(v * _L, _L)]
          mv = lax.shift_right_logical(yv, 9) & 31
          for u in range(_L):
            row = yv[u]
            keep = mv[u] == w

            @pl.when(keep)
            def _one():
              p = c * _YCHUNK + v * _L + u
              pltpu.async_copy(table_hbm.at[pl.ds(row, 1)],
                               wbuf.at[pl.ds(0, 1)], semw)
              pltpu.make_async_copy(table_hbm.at[pl.ds(0, 1)],
                                    wbuf.at[pl.ds(0, 1)], semw).wait()
              for t in range(WIDTH // _L):
                sb0[pl.ds(t * _L, _L)] = wbuf[0, pl.ds(t * _L, _L)]
                ibs[pl.ds(t * _L, _L)] = p * WIDTH + t * _L + lanes
              pltpu.async_copy(sb0.at[pl.ds(0, WIDTH)], out_hbm.at[ibs], ss0)
              pltpu.make_async_copy(out_hbm.at[pl.ds(0, WIDTH)],
                                    sb0.at[pl.ds(0, WIDTH)], ss0).wait()

          return _

        lax.fori_loop(0, _YCHUNK // _L, vec_body, 0)
        return _

      lax.fori_loop(0, BATCH // _YCHUNK, slow_chunk, 0)

  return gather_kernel


_gather = _make_gather()


@jax.jit
def kernel(y, genre_emb):
  out = _gather(y.astype(jnp.int32), genre_emb)
  return out.reshape(BATCH, 1, WIDTH)


# speedup vs baseline: 12.6391x; 12.6391x over previous
import functools

import jax
import jax.numpy as jnp
from jax import lax
from jax.experimental import pallas as pl
from jax.experimental.pallas import tpu as pltpu
from jax.experimental.pallas import tpu_sc as plsc

BATCH = 16384
WIDTH = 64
ROWS = 1000000

_info = plsc.get_sparse_core_info()
_NC, _NS, _L = _info.num_cores, _info.num_subcores, _info.num_lanes
_NW = _NC * _NS
_B_PER_W = BATCH // _NW
_NDMA = 192


def _make_gather():
  mesh = plsc.VectorSubcoreMesh(core_axis_name="c", subcore_axis_name="s")

  @functools.partial(
      pl.kernel,
      mesh=mesh,
      out_type=jax.ShapeDtypeStruct((BATCH, WIDTH), jnp.float32),
      scratch_types=[
          pltpu.VMEM((_B_PER_W,), jnp.int32),
          pltpu.VMEM((_B_PER_W - _NDMA, WIDTH), jnp.float32),
          pltpu.SemaphoreType.DMA,
          pltpu.SemaphoreType.DMA,
      ],
  )
  def gather_kernel(y_hbm, table_hbm, out_hbm, idx_v, rows_v, semd, sems):
    wid = lax.axis_index("s") * _NC + lax.axis_index("c")
    base = wid * _B_PER_W
    pltpu.sync_copy(y_hbm.at[pl.ds(base, _B_PER_W)], idx_v)

    for j in range(0, _B_PER_W, _L):
      v = idx_v[pl.ds(j, _L)]
      for u in range(_L):
        k = j + u
        row = v[u]
        if k < _NDMA:
          pltpu.async_copy(
              table_hbm.at[pl.ds(row, 1)],
              out_hbm.at[pl.ds(base + k, 1)],
              semd,
          )
        else:
          pltpu.async_copy(
              table_hbm.at[pl.ds(row, 1)],
              rows_v.at[pl.ds(k - _NDMA, 1)],
              sems,
          )

    pltpu.make_async_copy(
        table_hbm.at[pl.ds(0, _B_PER_W - _NDMA)], rows_v, sems
    ).wait()
    pltpu.sync_copy(rows_v,
                    out_hbm.at[pl.ds(base + _NDMA, _B_PER_W - _NDMA)])
    pltpu.make_async_copy(
        table_hbm.at[pl.ds(0, _NDMA)],
        out_hbm.at[pl.ds(base, _NDMA)],
        semd,
    ).wait()

  return gather_kernel


_gather = _make_gather()


@jax.jit
def kernel(y, genre_emb):
  out = _gather(y.astype(jnp.int32), genre_emb)
  return out.reshape(BATCH, 1, WIDTH)


# speedup vs baseline: 15.6024x; 1.2345x over previous
import functools

import jax
import jax.numpy as jnp
from jax import lax
from jax.experimental import pallas as pl
from jax.experimental.pallas import tpu as pltpu
from jax.experimental.pallas import tpu_sc as plsc

BATCH = 16384
WIDTH = 64
ROWS = 1000000

_info = plsc.get_sparse_core_info()
_NC, _NS, _L = _info.num_cores, _info.num_subcores, _info.num_lanes
_NW = _NC * _NS
_B_PER_W = BATCH // _NW


def _make_gather():
  mesh = plsc.VectorSubcoreMesh(core_axis_name="c", subcore_axis_name="s")

  @functools.partial(
      pl.kernel,
      mesh=mesh,
      out_type=jax.ShapeDtypeStruct((BATCH, WIDTH), jnp.float32),
      scratch_types=[
          pltpu.VMEM((_B_PER_W,), jnp.int32),
          pltpu.VMEM((_B_PER_W, WIDTH), jnp.float32),
          pltpu.SemaphoreType.DMA,
          pltpu.SemaphoreType.DMA,
      ],
  )
  def gather_kernel(y_hbm, table_hbm, out_hbm, idx_v, rows_v, sem0, sem1):
    wid = lax.axis_index("s") * _NC + lax.axis_index("c")
    base = wid * _B_PER_W
    pltpu.sync_copy(y_hbm.at[pl.ds(base, _B_PER_W)], idx_v)

    for j in range(0, _B_PER_W, _L):
      v = idx_v[pl.ds(j, _L)]
      for u in range(_L):
        k = j + u
        pltpu.async_copy(
            table_hbm.at[pl.ds(v[u], 1)],
            rows_v.at[pl.ds(k, 1)],
            sem0 if k % 2 == 0 else sem1,
        )

    pltpu.make_async_copy(
        table_hbm.at[pl.ds(0, _B_PER_W // 2)],
        rows_v.at[pl.ds(0, _B_PER_W // 2)],
        sem0,
    ).wait()
    pltpu.make_async_copy(
        table_hbm.at[pl.ds(0, _B_PER_W // 2)],
        rows_v.at[pl.ds(0, _B_PER_W // 2)],
        sem1,
    ).wait()

    pltpu.sync_copy(rows_v, out_hbm.at[pl.ds(base, _B_PER_W)])

  return gather_kernel


_gather = _make_gather()


@jax.jit
def kernel(y, genre_emb):
  out = _gather(y.astype(jnp.int32), genre_emb)
  return out.reshape(BATCH, 1, WIDTH)
